# Initial kernel scaffold; baseline (speedup 1.0000x reference)
import jax
import jax.numpy as jnp
from jax.experimental import pallas as pl


def kernel(x_c, x_d_list, x_o_list, t_emb, edge_index, edge_weight, node_types, wc, bc, Wcat, bcat, Word, bord, Wup, bup, woc, boc, Wocat, bocat, wood, bood):
    raise NotImplementedError("write your pallas kernel here")


# speedup vs baseline: 8.0510x; 8.0510x over previous
import jax
import jax.numpy as jnp
from jax.experimental import pallas as pl

B = 1024
D_C = 48
N_CAT = 16
CAT_DIM = 10
N_ORD = 16
D = D_C + N_CAT + N_ORD
HD = 64
TED = 64
NL = 3
NT = 3
E = 1280
CIN = 2 * HD + TED

F32 = jnp.float32


def _dot(a, b, dims=((1,), (0,))):
    return jax.lax.dot_general(a, b, (dims, ((), ())),
                               preferred_element_type=F32)


def _gelu(z):
    return 0.5 * z * (1.0 + jax.lax.erf(z * 0.7071067811865476))


def _full(s):
    return pl.BlockSpec(s, lambda *_: (0,) * len(s))


def _adj_kernel(ei_ref, ew_ref, an_ref):
    src = ei_ref[0:1, :]
    dst = ei_ref[1:2, :]
    w = ew_ref[...]
    iota_de = jax.lax.broadcasted_iota(jnp.int32, (D, E), 0)
    md = jnp.where(iota_de == dst, w, 0.0)
    ms = (iota_de == src).astype(F32)
    A = _dot(md, ms, ((1,), (1,)))
    deg = jnp.sum(md, axis=1, keepdims=True)
    an_ref[...] = A / jnp.maximum(deg, 1e-8)


BB_P = 256


def _proj_kernel(xcT_ref, xd_ref, xoT_ref, wc_ref, bc_ref, Wcat_ref,
                 bcat_ref, Word_ref, bord_ref, h_ref):
    emb_c = xcT_ref[...] * wc_ref[...] + bc_ref[...]
    cat_parts = []
    for k in range(N_CAT):
        yk = _dot(xd_ref[k], Wcat_ref[k])
        cat_parts.append((yk + bcat_ref[k:k + 1, :])[None])
    emb_cat = jnp.concatenate(cat_parts, axis=0)
    xo = xoT_ref[...]
    w0 = Word_ref[0:1, :][None]
    w1 = Word_ref[1:2, :][None]
    emb_o = jnp.cos(xo) * w0 + jnp.sin(xo) * w1 + bord_ref[...][None]
    h_ref[...] = jnp.concatenate([emb_c, emb_cat, emb_o], axis=0)


LB = 8192


def _agg_kernel(an_ref, ha_ref, out_ref):
    out_ref[...] = _dot(an_ref[...], ha_ref[...])


DB = 8


def _mlp_kernel(h_ref, agg_ref, t_ref, nt_ref, W_ref, b_ref, out_ref):
    h2 = h_ref[...].reshape(DB * B, HD)
    a2 = agg_ref[...].reshape(DB * B, HD)
    t_emb = t_ref[...]
    nt = nt_ref[...]
    p_sel = jnp.zeros((DB, B, HD), F32)
    for t in range(NT):
        W = W_ref[t]
        p = _dot(h2, W[0:HD]) + _dot(a2, W[HD:2 * HD])
        tc = _dot(t_emb, W[2 * HD:CIN]) + b_ref[t:t + 1, :]
        mask = (nt == t).astype(F32)
        p_sel = p_sel + (p.reshape(DB, B, HD) + tc[None]) * mask
    out_ref[...] = h_ref[...] + _gelu(p_sel)


BB_H = 256


def _heads_kernel(h_ref, woc_ref, boc_ref, Wocat_ref, bocat_ref,
                  wood_ref, bood_ref, vcT_ref, vd_ref, vo_ref):
    h = h_ref[...]
    vcT = jnp.sum(h[0:D_C] * woc_ref[...], axis=-1)
    vcT_ref[...] = vcT + boc_ref[...]
    for k in range(N_CAT):
        yk = _dot(h[D_C + k], Wocat_ref[k])
        yk = yk + bocat_ref[k:k + 1, :]
        vd_ref[k] = yk - jnp.mean(yk, axis=-1, keepdims=True)
    vo = jnp.sum(h[D_C + N_CAT:D] * wood_ref[...], axis=-1)
    vo_ref[...] = vo + bood_ref[...]


def kernel(x_c, x_d_list, x_o_list, t_emb, edge_index, edge_weight,
           node_types, wc, bc, Wcat, bcat, Word, bord, Wup, bup,
           woc, boc, Wocat, bocat, wood, bood):
    xcT = x_c.T[:, :, None]
    xoT = x_o_list[:, :, None]
    ew2 = edge_weight[None, :]
    nt3 = node_types[:, None, None]
    wc3 = wc[None, None, :]
    bc3 = bc[None, None, :]
    bord2 = bord[None, :]
    woc3 = woc[None, None, :]
    wood3 = wood[None, None, :]
    boc2 = boc[None, :]
    bood2 = bood[None, :]

    An = pl.pallas_call(
        _adj_kernel,
        in_specs=[_full((2, E)), _full((1, E))],
        out_specs=_full((D, D)),
        out_shape=jax.ShapeDtypeStruct((D, D), F32),
    )(edge_index, ew2)

    h = pl.pallas_call(
        _proj_kernel,
        grid=(B // BB_P,),
        in_specs=[
            pl.BlockSpec((D_C, BB_P, 1), lambda i: (0, i, 0)),
            pl.BlockSpec((N_CAT, BB_P, CAT_DIM), lambda i: (0, i, 0)),
            pl.BlockSpec((N_ORD, BB_P, 1), lambda i: (0, i, 0)),
            _full((1, 1, HD)), _full((1, 1, HD)),
            _full((N_CAT, CAT_DIM, HD)), _full((N_CAT, HD)),
            _full((2, HD)), _full((1, HD)),
        ],
        out_specs=pl.BlockSpec((D, BB_P, HD), lambda i: (0, i, 0)),
        out_shape=jax.ShapeDtypeStruct((D, B, HD), F32),
    )(xcT, x_d_list, xoT, wc3, bc3, Wcat, bcat, Word, bord2)

    agg_call = pl.pallas_call(
        _agg_kernel,
        grid=(B * HD // LB,),
        in_specs=[_full((D, D)), pl.BlockSpec((D, LB), lambda i: (0, i))],
        out_specs=pl.BlockSpec((D, LB), lambda i: (0, i)),
        out_shape=jax.ShapeDtypeStruct((D, B * HD), F32),
    )
    mlp_call = pl.pallas_call(
        _mlp_kernel,
        grid=(D // DB,),
        in_specs=[
            pl.BlockSpec((DB, B, HD), lambda i: (i, 0, 0)),
            pl.BlockSpec((DB, B, HD), lambda i: (i, 0, 0)),
            _full((B, TED)),
            pl.BlockSpec((DB, 1, 1), lambda i: (i, 0, 0)),
            _full((NT, CIN, HD)), _full((NT, HD)),
        ],
        out_specs=pl.BlockSpec((DB, B, HD), lambda i: (i, 0, 0)),
        out_shape=jax.ShapeDtypeStruct((D, B, HD), F32),
    )
    for l in range(NL):
        agg = agg_call(An, h.reshape(D, B * HD)).reshape(D, B, HD)
        h = mlp_call(h, agg, t_emb, nt3, Wup[l], bup[l])

    vcT, v_d, v_o = pl.pallas_call(
        _heads_kernel,
        grid=(B // BB_H,),
        in_specs=[
            pl.BlockSpec((D, BB_H, HD), lambda i: (0, i, 0)),
            _full((1, 1, HD)), _full((1, 1)),
            _full((N_CAT, HD, CAT_DIM)), _full((N_CAT, CAT_DIM)),
            _full((1, 1, HD)), _full((1, 1)),
        ],
        out_specs=[
            pl.BlockSpec((D_C, BB_H), lambda i: (0, i)),
            pl.BlockSpec((N_CAT, BB_H, CAT_DIM), lambda i: (0, i, 0)),
            pl.BlockSpec((N_ORD, BB_H), lambda i: (0, i)),
        ],
        out_shape=[
            jax.ShapeDtypeStruct((D_C, B), F32),
            jax.ShapeDtypeStruct((N_CAT, B, CAT_DIM), F32),
            jax.ShapeDtypeStruct((N_ORD, B), F32),
        ],
    )(h, woc3, boc2, Wocat, bocat, wood3, bood2)
    return vcT.T, v_d, v_o


# speedup vs baseline: 11.4137x; 1.4177x over previous
import jax
import jax.numpy as jnp
from jax.experimental import pallas as pl

B = 1024
D_C = 48
N_CAT = 16
CAT_DIM = 10
N_ORD = 16
D = D_C + N_CAT + N_ORD
HD = 64
TED = 64
NL = 3
NT = 3
E = 1280
CIN = 2 * HD + TED

F32 = jnp.float32


def _dot(a, b, dims=((1,), (0,))):
    return jax.lax.dot_general(a, b, (dims, ((), ())),
                               preferred_element_type=F32)


def _gelu(z):
    return 0.5 * z * (1.0 + jax.lax.erf(z * 0.7071067811865476))


def _full(s):
    return pl.BlockSpec(s, lambda *_: (0,) * len(s))


def _adj_kernel(ei_ref, ew_ref, an_ref):
    src = ei_ref[0:1, :]
    dst = ei_ref[1:2, :]
    w = ew_ref[...]
    iota_de = jax.lax.broadcasted_iota(jnp.int32, (D, E), 0)
    md = jnp.where(iota_de == dst, w, 0.0)
    ms = (iota_de == src).astype(F32)
    A = _dot(md, ms, ((1,), (1,)))
    deg = jnp.sum(md, axis=1, keepdims=True)
    an_ref[...] = A / jnp.maximum(deg, 1e-8)


BB_P = 256


def _proj_kernel(xcT_ref, xd_ref, xo_ref, wc_ref, bc_ref, Wcat_ref,
                 bcat_ref, WordT_ref, bord_ref, h_ref):
    h_ref[0:D_C] = xcT_ref[...] * wc_ref[...] + bc_ref[...]
    for k in range(N_CAT):
        yk = _dot(xd_ref[k], Wcat_ref[k])
        h_ref[D_C + k] = yk + bcat_ref[k:k + 1, :]
    xo = xo_ref[...]
    c = jnp.cos(xo)
    s = jnp.sin(xo)
    WT = WordT_ref[...]
    for k in range(N_ORD):
        cs = jnp.concatenate([c[k:k + 1, :], s[k:k + 1, :]], axis=0)
        ek = _dot(WT, cs)
        h_ref[D_C + N_CAT + k] = ek.T + bord_ref[...]


BB_A = 128


def _agg_kernel(an_ref, h_ref, out_ref):
    out_ref[...] = _dot(an_ref[...], h_ref[...])


DB = 8


def _mlp_kernel(h_ref, agg_ref, t_ref, nt_ref, W_ref, b_ref, out_ref):
    h2 = h_ref[...].reshape(DB * B, HD)
    a2 = agg_ref[...].reshape(DB * B, HD)
    W = W_ref[...]
    p_all = _dot(h2, W[0:HD]) + _dot(a2, W[HD:2 * HD])
    tc_all = _dot(t_ref[...], W[2 * HD:CIN]) + b_ref[...]
    nt = nt_ref[...]
    p_sel = jnp.zeros((DB, B, HD), F32)
    for t in range(NT):
        p = p_all[:, t * HD:(t + 1) * HD].reshape(DB, B, HD)
        tc = tc_all[:, t * HD:(t + 1) * HD]
        mask = (nt == t).astype(F32)
        p_sel = p_sel + (p + tc[None]) * mask
    out_ref[...] = h_ref[...] + _gelu(p_sel)


BB_H = 256


def _heads_kernel(h_ref, woc_ref, boc_ref, Wocat_ref, bocat_ref,
                  wood_ref, bood_ref, vcT_ref, vd_ref, vo_ref):
    h = h_ref[...]
    vcT = jnp.sum(h[0:D_C] * woc_ref[...], axis=-1)
    vcT_ref[...] = vcT + boc_ref[...]
    for k in range(N_CAT):
        yk = _dot(h[D_C + k], Wocat_ref[k])
        yk = yk + bocat_ref[k:k + 1, :]
        vd_ref[k] = yk - jnp.mean(yk, axis=-1, keepdims=True)
    vo = jnp.sum(h[D_C + N_CAT:D] * wood_ref[...], axis=-1)
    vo_ref[...] = vo + bood_ref[...]


def kernel(x_c, x_d_list, x_o_list, t_emb, edge_index, edge_weight,
           node_types, wc, bc, Wcat, bcat, Word, bord, Wup, bup,
           woc, boc, Wocat, bocat, wood, bood):
    xcT = x_c.T[:, :, None]
    xoT = x_o_list[:, :, None]
    ew2 = edge_weight[None, :]
    nt3 = node_types[:, None, None]
    wc3 = wc[None, None, :]
    bc3 = bc[None, None, :]
    bord2 = bord[None, :]
    woc3 = woc[None, None, :]
    wood3 = wood[None, None, :]
    boc2 = boc[None, :]
    bood2 = bood[None, :]

    An = pl.pallas_call(
        _adj_kernel,
        in_specs=[_full((2, E)), _full((1, E))],
        out_specs=_full((D, D)),
        out_shape=jax.ShapeDtypeStruct((D, D), F32),
    )(edge_index, ew2)

    h = pl.pallas_call(
        _proj_kernel,
        grid=(B // BB_P,),
        in_specs=[
            pl.BlockSpec((D_C, BB_P, 1), lambda i: (0, i, 0)),
            pl.BlockSpec((N_CAT, BB_P, CAT_DIM), lambda i: (0, i, 0)),
            pl.BlockSpec((N_ORD, BB_P), lambda i: (0, i)),
            _full((1, 1, HD)), _full((1, 1, HD)),
            _full((N_CAT, CAT_DIM, HD)), _full((N_CAT, HD)),
            _full((HD, 2)), _full((1, HD)),
        ],
        out_specs=pl.BlockSpec((D, BB_P, HD), lambda i: (0, i, 0)),
        out_shape=jax.ShapeDtypeStruct((D, B, HD), F32),
    )(xcT, x_d_list, x_o_list, wc3, bc3, Wcat, bcat, Word.T, bord2)

    agg_call = pl.pallas_call(
        _agg_kernel,
        grid=(B // BB_A,),
        in_specs=[_full((D, D)),
                  pl.BlockSpec((D, BB_A, HD), lambda i: (0, i, 0))],
        out_specs=pl.BlockSpec((D, BB_A, HD), lambda i: (0, i, 0)),
        out_shape=jax.ShapeDtypeStruct((D, B, HD), F32),
    )
    mlp_call = pl.pallas_call(
        _mlp_kernel,
        grid=(D // DB,),
        in_specs=[
            pl.BlockSpec((DB, B, HD), lambda i: (i, 0, 0)),
            pl.BlockSpec((DB, B, HD), lambda i: (i, 0, 0)),
            _full((B, TED)),
            pl.BlockSpec((DB, 1, 1), lambda i: (i, 0, 0)),
            _full((CIN, NT * HD)), _full((1, NT * HD)),
        ],
        out_specs=pl.BlockSpec((DB, B, HD), lambda i: (i, 0, 0)),
        out_shape=jax.ShapeDtypeStruct((D, B, HD), F32),
    )
    Wpack = jnp.transpose(Wup, (0, 2, 1, 3)).reshape(NL, CIN, NT * HD)
    bpack = bup.reshape(NL, 1, NT * HD)
    for l in range(NL):
        agg = agg_call(An, h)
        h = mlp_call(h, agg, t_emb, nt3, Wpack[l], bpack[l])

    vcT, v_d, v_o = pl.pallas_call(
        _heads_kernel,
        grid=(B // BB_H,),
        in_specs=[
            pl.BlockSpec((D, BB_H, HD), lambda i: (0, i, 0)),
            _full((1, 1, HD)), _full((1, 1)),
            _full((N_CAT, HD, CAT_DIM)), _full((N_CAT, CAT_DIM)),
            _full((1, 1, HD)), _full((1, 1)),
        ],
        out_specs=[
            pl.BlockSpec((D_C, BB_H), lambda i: (0, i)),
            pl.BlockSpec((N_CAT, BB_H, CAT_DIM), lambda i: (0, i, 0)),
            pl.BlockSpec((N_ORD, BB_H), lambda i: (0, i)),
        ],
        out_shape=[
            jax.ShapeDtypeStruct((D_C, B), F32),
            jax.ShapeDtypeStruct((N_CAT, B, CAT_DIM), F32),
            jax.ShapeDtypeStruct((N_ORD, B), F32),
        ],
    )(h, woc3, boc2, Wocat, bocat, wood3, bood2)
    return vcT.T, v_d, v_o


# speedup vs baseline: 13.3510x; 1.1697x over previous
import jax
import jax.numpy as jnp
from jax.experimental import pallas as pl

B = 1024
D_C = 48
N_CAT = 16
CAT_DIM = 10
N_ORD = 16
D = D_C + N_CAT + N_ORD
HD = 64
TED = 64
NL = 3
NT = 3
E = 1280
CIN = 2 * HD + TED

F32 = jnp.float32


def _dot(a, b, dims=((1,), (0,))):
    return jax.lax.dot_general(a, b, (dims, ((), ())),
                               preferred_element_type=F32)


def _gelu(z):
    return 0.5 * z * (1.0 + jax.lax.erf(z * 0.7071067811865476))


def _full(s):
    return pl.BlockSpec(s, lambda *_: (0,) * len(s))


def _adj_kernel(ei_ref, ew_ref, an_ref):
    src = ei_ref[0:1, :]
    dst = ei_ref[1:2, :]
    w = ew_ref[...]
    iota_de = jax.lax.broadcasted_iota(jnp.int32, (D, E), 0)
    md = jnp.where(iota_de == dst, w, 0.0)
    ms = (iota_de == src).astype(F32)
    A = _dot(md, ms, ((1,), (1,)))
    deg = jnp.sum(md, axis=1, keepdims=True)
    an_ref[...] = A / jnp.maximum(deg, 1e-8)


BB_P = 256


def _proj_kernel(xcT_ref, xd_ref, xo_ref, wc_ref, bc_ref, Wcat_ref,
                 bcat_ref, WordT_ref, bord_ref, h_ref):
    h_ref[0:D_C] = xcT_ref[...] * wc_ref[...] + bc_ref[...]
    for k in range(N_CAT):
        yk = _dot(xd_ref[k], Wcat_ref[k])
        h_ref[D_C + k] = yk + bcat_ref[k:k + 1, :]
    xo = xo_ref[...]
    c = jnp.cos(xo)
    s = jnp.sin(xo)
    WT = WordT_ref[...]
    for k in range(N_ORD):
        cs = jnp.concatenate([c[k:k + 1, :], s[k:k + 1, :]], axis=0)
        ek = _dot(WT, cs)
        h_ref[D_C + N_CAT + k] = ek.T + bord_ref[...]


BB_L = 128


def _layer_kernel(an_ref, h_ref, t_ref, nt_ref, W_ref, b_ref, out_ref):
    hb = h_ref[...]
    agg = _dot(an_ref[...], hb)
    h2 = hb.reshape(D * BB_L, HD)
    a2 = agg.reshape(D * BB_L, HD)
    W = W_ref[...]
    p_all = _dot(h2, W[0:HD]) + _dot(a2, W[HD:2 * HD])
    tc_all = _dot(t_ref[...], W[2 * HD:CIN]) + b_ref[...]
    nt = nt_ref[...]
    p_sel = jnp.zeros((D, BB_L, HD), F32)
    for t in range(NT):
        p = p_all[:, t * HD:(t + 1) * HD].reshape(D, BB_L, HD)
        tc = tc_all[:, t * HD:(t + 1) * HD]
        mask = (nt == t).astype(F32)
        p_sel = p_sel + (p + tc[None]) * mask
    out_ref[...] = hb + _gelu(p_sel)


BB_H = 256


def _heads_kernel(h_ref, woc_ref, boc_ref, Wocat_ref, bocat_ref,
                  wood_ref, bood_ref, vcT_ref, vd_ref, vo_ref):
    h = h_ref[...]
    vcT = jnp.sum(h[0:D_C] * woc_ref[...], axis=-1)
    vcT_ref[...] = vcT + boc_ref[...]
    for k in range(N_CAT):
        yk = _dot(h[D_C + k], Wocat_ref[k])
        yk = yk + bocat_ref[k:k + 1, :]
        vd_ref[k] = yk - jnp.mean(yk, axis=-1, keepdims=True)
    vo = jnp.sum(h[D_C + N_CAT:D] * wood_ref[...], axis=-1)
    vo_ref[...] = vo + bood_ref[...]


def kernel(x_c, x_d_list, x_o_list, t_emb, edge_index, edge_weight,
           node_types, wc, bc, Wcat, bcat, Word, bord, Wup, bup,
           woc, boc, Wocat, bocat, wood, bood):
    xcT = x_c.T[:, :, None]
    xoT = x_o_list[:, :, None]
    ew2 = edge_weight[None, :]
    nt3 = node_types[:, None, None]
    wc3 = wc[None, None, :]
    bc3 = bc[None, None, :]
    bord2 = bord[None, :]
    woc3 = woc[None, None, :]
    wood3 = wood[None, None, :]
    boc2 = boc[None, :]
    bood2 = bood[None, :]

    An = pl.pallas_call(
        _adj_kernel,
        in_specs=[_full((2, E)), _full((1, E))],
        out_specs=_full((D, D)),
        out_shape=jax.ShapeDtypeStruct((D, D), F32),
    )(edge_index, ew2)

    h = pl.pallas_call(
        _proj_kernel,
        grid=(B // BB_P,),
        in_specs=[
            pl.BlockSpec((D_C, BB_P, 1), lambda i: (0, i, 0)),
            pl.BlockSpec((N_CAT, BB_P, CAT_DIM), lambda i: (0, i, 0)),
            pl.BlockSpec((N_ORD, BB_P), lambda i: (0, i)),
            _full((1, 1, HD)), _full((1, 1, HD)),
            _full((N_CAT, CAT_DIM, HD)), _full((N_CAT, HD)),
            _full((HD, 2)), _full((1, HD)),
        ],
        out_specs=pl.BlockSpec((D, BB_P, HD), lambda i: (0, i, 0)),
        out_shape=jax.ShapeDtypeStruct((D, B, HD), F32),
    )(xcT, x_d_list, x_o_list, wc3, bc3, Wcat, bcat, Word.T, bord2)

    layer_call = pl.pallas_call(
        _layer_kernel,
        grid=(B // BB_L,),
        in_specs=[
            _full((D, D)),
            pl.BlockSpec((D, BB_L, HD), lambda i: (0, i, 0)),
            pl.BlockSpec((BB_L, TED), lambda i: (i, 0)),
            _full((D, 1, 1)),
            _full((CIN, NT * HD)), _full((1, NT * HD)),
        ],
        out_specs=pl.BlockSpec((D, BB_L, HD), lambda i: (0, i, 0)),
        out_shape=jax.ShapeDtypeStruct((D, B, HD), F32),
    )
    Wpack = jnp.transpose(Wup, (0, 2, 1, 3)).reshape(NL, CIN, NT * HD)
    bpack = bup.reshape(NL, 1, NT * HD)
    for l in range(NL):
        h = layer_call(An, h, t_emb, nt3, Wpack[l], bpack[l])

    vcT, v_d, v_o = pl.pallas_call(
        _heads_kernel,
        grid=(B // BB_H,),
        in_specs=[
            pl.BlockSpec((D, BB_H, HD), lambda i: (0, i, 0)),
            _full((1, 1, HD)), _full((1, 1)),
            _full((N_CAT, HD, CAT_DIM)), _full((N_CAT, CAT_DIM)),
            _full((1, 1, HD)), _full((1, 1)),
        ],
        out_specs=[
            pl.BlockSpec((D_C, BB_H), lambda i: (0, i)),
            pl.BlockSpec((N_CAT, BB_H, CAT_DIM), lambda i: (0, i, 0)),
            pl.BlockSpec((N_ORD, BB_H), lambda i: (0, i)),
        ],
        out_shape=[
            jax.ShapeDtypeStruct((D_C, B), F32),
            jax.ShapeDtypeStruct((N_CAT, B, CAT_DIM), F32),
            jax.ShapeDtypeStruct((N_ORD, B), F32),
        ],
    )(h, woc3, boc2, Wocat, bocat, wood3, bood2)
    return vcT.T, v_d, v_o


# speedup vs baseline: 13.4311x; 1.0060x over previous
import jax
import jax.numpy as jnp
from jax.experimental import pallas as pl

B = 1024
D_C = 48
N_CAT = 16
CAT_DIM = 10
N_ORD = 16
D = D_C + N_CAT + N_ORD
HD = 64
TED = 64
NL = 3
NT = 3
E = 1280
CIN = 2 * HD + TED

F32 = jnp.float32
BF16 = jnp.bfloat16


def _dot(a, b, dims=((1,), (0,))):
    return jax.lax.dot_general(a, b, (dims, ((), ())),
                               preferred_element_type=F32)


def _gelu(z):
    return 0.5 * z * (1.0 + jax.lax.erf(z * 0.7071067811865476))


def _full(s):
    return pl.BlockSpec(s, lambda *_: (0,) * len(s))


BB_P = 256


def _proj_kernel(xcT_ref, xd_ref, xo_ref, ei_ref, ew_ref, wc_ref, bc_ref,
                 Wcat_ref, bcat_ref, WordT_ref, bord_ref, h_ref, an_ref):
    @pl.when(pl.program_id(0) == 0)
    def _build_adjacency():
        src = ei_ref[0:1, :]
        dst = ei_ref[1:2, :]
        w = ew_ref[...]
        iota_de = jax.lax.broadcasted_iota(jnp.int32, (D, E), 0)
        md = jnp.where(iota_de == dst, w, 0.0)
        ms = (iota_de == src).astype(F32)
        A = _dot(md, ms, ((1,), (1,)))
        deg = jnp.sum(md, axis=1, keepdims=True)
        an_ref[...] = A / jnp.maximum(deg, 1e-8)

    h_ref[0:D_C] = xcT_ref[...] * wc_ref[...] + bc_ref[...]
    for k in range(N_CAT):
        yk = _dot(xd_ref[k], Wcat_ref[k])
        h_ref[D_C + k] = yk + bcat_ref[k:k + 1, :]
    xo = xo_ref[...]
    c = jnp.cos(xo)
    s = jnp.sin(xo)
    WT = WordT_ref[...]
    for k in range(N_ORD):
        cs = jnp.concatenate([c[k:k + 1, :], s[k:k + 1, :]], axis=0)
        ek = _dot(WT, cs)
        h_ref[D_C + N_CAT + k] = ek.T + bord_ref[...]


BB_L = 128


def _layer_update(an_ref, hb, t_ref, nt_ref, W_ref, b_ref):
    agg = _dot(an_ref[...], hb)
    h2 = hb.reshape(D * BB_L, HD)
    a2 = agg.reshape(D * BB_L, HD)
    W = W_ref[...]
    p_all = _dot(h2, W[0:HD]) + _dot(a2, W[HD:2 * HD])
    tc_all = _dot(t_ref[...], W[2 * HD:CIN]) + b_ref[...]
    nt = nt_ref[...]
    p_sel = jnp.zeros((D, BB_L, HD), F32)
    for t in range(NT):
        p = p_all[:, t * HD:(t + 1) * HD].reshape(D, BB_L, HD)
        tc = tc_all[:, t * HD:(t + 1) * HD]
        mask = (nt == t).astype(F32)
        p_sel = p_sel + (p + tc[None]) * mask
    return hb + _gelu(p_sel)


def _layer_kernel(an_ref, h_ref, t_ref, nt_ref, W_ref, b_ref, out_ref):
    out_ref[...] = _layer_update(an_ref, h_ref[...], t_ref, nt_ref,
                                 W_ref, b_ref)


def _last_kernel(an_ref, h_ref, t_ref, nt_ref, W_ref, b_ref,
                 woc_ref, boc_ref, Wocat_ref, bocat_ref, wood_ref, bood_ref,
                 vcT_ref, vd_ref, vo_ref):
    h = _layer_update(an_ref, h_ref[...], t_ref, nt_ref, W_ref, b_ref)
    vcT = jnp.sum(h[0:D_C] * woc_ref[...], axis=-1)
    vcT_ref[...] = vcT + boc_ref[...]
    for k in range(N_CAT):
        yk = _dot(h[D_C + k], Wocat_ref[k])
        yk = yk + bocat_ref[k:k + 1, :]
        vd_ref[k] = yk - jnp.mean(yk, axis=-1, keepdims=True)
    vo = jnp.sum(h[D_C + N_CAT:D] * wood_ref[...], axis=-1)
    vo_ref[...] = vo + bood_ref[...]


def kernel(x_c, x_d_list, x_o_list, t_emb, edge_index, edge_weight,
           node_types, wc, bc, Wcat, bcat, Word, bord, Wup, bup,
           woc, boc, Wocat, bocat, wood, bood):
    xcT = x_c.T[:, :, None]
    ew2 = edge_weight[None, :]
    nt3 = node_types[:, None, None]
    wc3 = wc[None, None, :]
    bc3 = bc[None, None, :]
    bord2 = bord[None, :]
    woc3 = woc[None, None, :]
    wood3 = wood[None, None, :]
    boc2 = boc[None, :]
    bood2 = bood[None, :]

    h, An = pl.pallas_call(
        _proj_kernel,
        grid=(B // BB_P,),
        in_specs=[
            pl.BlockSpec((D_C, BB_P, 1), lambda i: (0, i, 0)),
            pl.BlockSpec((N_CAT, BB_P, CAT_DIM), lambda i: (0, i, 0)),
            pl.BlockSpec((N_ORD, BB_P), lambda i: (0, i)),
            _full((2, E)), _full((1, E)),
            _full((1, 1, HD)), _full((1, 1, HD)),
            _full((N_CAT, CAT_DIM, HD)), _full((N_CAT, HD)),
            _full((HD, 2)), _full((1, HD)),
        ],
        out_specs=[
            pl.BlockSpec((D, BB_P, HD), lambda i: (0, i, 0)),
            _full((D, D)),
        ],
        out_shape=[
            jax.ShapeDtypeStruct((D, B, HD), F32),
            jax.ShapeDtypeStruct((D, D), F32),
        ],
    )(xcT, x_d_list, x_o_list, edge_index, ew2, wc3, bc3, Wcat, bcat,
      Word.T, bord2)

    Wpack = jnp.transpose(Wup, (0, 2, 1, 3)).reshape(NL, CIN, NT * HD)
    bpack = bup.reshape(NL, 1, NT * HD)

    layer_in_specs = [
        _full((D, D)),
        pl.BlockSpec((D, BB_L, HD), lambda i: (0, i, 0)),
        pl.BlockSpec((BB_L, TED), lambda i: (i, 0)),
        _full((D, 1, 1)),
        _full((CIN, NT * HD)), _full((1, NT * HD)),
    ]
    layer_call = pl.pallas_call(
        _layer_kernel,
        grid=(B // BB_L,),
        in_specs=layer_in_specs,
        out_specs=pl.BlockSpec((D, BB_L, HD), lambda i: (0, i, 0)),
        out_shape=jax.ShapeDtypeStruct((D, B, HD), F32),
    )
    for l in range(NL - 1):
        h = layer_call(An, h, t_emb, nt3, Wpack[l], bpack[l])

    vcT, v_d, v_o = pl.pallas_call(
        _last_kernel,
        grid=(B // BB_L,),
        in_specs=layer_in_specs + [
            _full((1, 1, HD)), _full((1, 1)),
            _full((N_CAT, HD, CAT_DIM)), _full((N_CAT, CAT_DIM)),
            _full((1, 1, HD)), _full((1, 1)),
        ],
        out_specs=[
            pl.BlockSpec((D_C, BB_L), lambda i: (0, i)),
            pl.BlockSpec((N_CAT, BB_L, CAT_DIM), lambda i: (0, i, 0)),
            pl.BlockSpec((N_ORD, BB_L), lambda i: (0, i)),
        ],
        out_shape=[
            jax.ShapeDtypeStruct((D_C, B), F32),
            jax.ShapeDtypeStruct((N_CAT, B, CAT_DIM), F32),
            jax.ShapeDtypeStruct((N_ORD, B), F32),
        ],
    )(An, h, t_emb, nt3, Wpack[NL - 1], bpack[NL - 1],
      woc3, boc2, Wocat, bocat, wood3, bood2)
    return vcT.T, v_d, v_o
